# Initial kernel scaffold; baseline (speedup 1.0000x reference)
import jax
import jax.numpy as jnp
from jax.experimental import pallas as pl


def kernel(field_ids, values, field_table, scale_table):
    raise NotImplementedError("write your pallas kernel here")


# speedup vs baseline: 2.9651x; 2.9651x over previous
import functools

import jax
import jax.numpy as jnp
from jax import lax
from jax.experimental import pallas as pl
from jax.experimental.pallas import tpu as pltpu
from jax.experimental.pallas import tpu_sc as plsc

N = 106496
NUM_FIELDS = 100
D = 64
NUM_SCALES = 5
R = NUM_SCALES + 1
NW = 32
CHUNK = N // NW
SUB = 416
NSUB = CHUNK // SUB
GROUPS = SUB // 16
LANES = 16

_SF = [0.01, 0.1, 1.0, 10.0, 100.0]


def _body(table_hbm, ids_hbm, vals_hbm, out_hbm, table_v, ids_v, vals_v, out_v):
    wid = lax.axis_index("s") * 2 + lax.axis_index("c")
    base_tok = wid * CHUNK
    pltpu.sync_copy(table_hbm, table_v)
    pltpu.sync_copy(ids_hbm.at[pl.ds(base_tok, CHUNK)], ids_v)
    pltpu.sync_copy(vals_hbm.at[pl.ds(base_tok, CHUNK)], vals_v)
    lane = lax.iota(jnp.int32, LANES)

    def grp_body(s, g, _):
        t0 = s * SUB + g * LANES
        fvec = ids_v[pl.ds(t0, LANES)]
        vvec = vals_v[pl.ds(t0, LANES)]
        base = fvec * (R * D)
        ws = []
        for r in range(NUM_SCALES):
            x = vvec * (2.0 * _SF[r])
            ws.append(1.0 - 2.0 / (jnp.exp(x) + 1.0))
        sbase = (g * LANES + lane) * D
        for d in range(D):
            acc = plsc.load_gather(table_v, [base + (NUM_SCALES * D + d)])
            for r in range(NUM_SCALES):
                acc = acc + ws[r] * plsc.load_gather(table_v, [base + (r * D + d)])
            plsc.store_scatter(out_v, [sbase + d], acc)
        return _

    def sub_body(s, _):
        lax.fori_loop(0, GROUPS, functools.partial(grp_body, s), 0)
        pltpu.sync_copy(out_v, out_hbm.at[pl.ds((base_tok + s * SUB) * D, SUB * D)])
        return _

    lax.fori_loop(0, NSUB, sub_body, 0)


def kernel(field_ids, values, field_table, scale_table):
    scale3 = scale_table.reshape(NUM_FIELDS, NUM_SCALES, D)
    combined = jnp.concatenate([scale3, field_table[:, None, :]], axis=1).reshape(-1)
    ids = field_ids.astype(jnp.int32)
    mesh = plsc.VectorSubcoreMesh(core_axis_name="c", subcore_axis_name="s")
    k = pl.kernel(
        _body,
        out_type=jax.ShapeDtypeStruct((N * D,), jnp.float32),
        mesh=mesh,
        compiler_params=pltpu.CompilerParams(needs_layout_passes=False),
        scratch_types=[
            pltpu.VMEM((NUM_FIELDS * R * D,), jnp.float32),
            pltpu.VMEM((CHUNK,), jnp.int32),
            pltpu.VMEM((CHUNK,), jnp.float32),
            pltpu.VMEM((SUB * D,), jnp.float32),
        ],
    )
    out = k(combined, ids, values)
    return out.reshape(N, D)


# speedup vs baseline: 7.7314x; 2.6075x over previous
import functools

import jax
import jax.numpy as jnp
from jax import lax
from jax.experimental import pallas as pl
from jax.experimental.pallas import tpu as pltpu
from jax.experimental.pallas import tpu_sc as plsc

N = 106496
NUM_FIELDS = 100
D = 64
NUM_SCALES = 5
R = NUM_SCALES + 1
NW = 32
CHUNK = N // NW
SUB = 416
NSUB = CHUNK // SUB
GROUPS = SUB // 16
LANES = 16

_SF = [0.01, 0.1, 1.0, 10.0, 100.0]
STRIDE = R * D + 1


def _body(table_hbm, ids_hbm, vals_hbm, out_hbm, table_v, ids_v, vals_v, out_v):
    wid = lax.axis_index("s") * 2 + lax.axis_index("c")
    base_tok = wid * CHUNK
    pltpu.sync_copy(table_hbm, table_v)
    pltpu.sync_copy(ids_hbm.at[pl.ds(base_tok, CHUNK)], ids_v)
    pltpu.sync_copy(vals_hbm.at[pl.ds(base_tok, CHUNK)], vals_v)
    lane = lax.iota(jnp.int32, LANES)

    def grp_body(s, g, _):
        t0 = s * SUB + g * LANES
        fvec = ids_v[pl.ds(t0, LANES)]
        vvec = vals_v[pl.ds(t0, LANES)]
        base = fvec * STRIDE
        ws = []
        for r in range(NUM_SCALES):
            x = vvec * (2.0 * _SF[r])
            ws.append(1.0 - 2.0 / (jnp.exp(x) + 1.0))
        sbase = (g * LANES + lane) * D
        for d in range(D):
            acc = plsc.load_gather(table_v, [base + (NUM_SCALES * D + d)])
            for r in range(NUM_SCALES):
                acc = acc + ws[r] * plsc.load_gather(table_v, [base + (r * D + d)])
            plsc.store_scatter(out_v, [sbase + d], acc)
        return _

    def sub_body(s, _):
        lax.fori_loop(0, GROUPS, functools.partial(grp_body, s), 0)
        pltpu.sync_copy(out_v, out_hbm.at[pl.ds((base_tok + s * SUB) * D, SUB * D)])
        return _

    lax.fori_loop(0, NSUB, sub_body, 0)


def kernel(field_ids, values, field_table, scale_table):
    scale2 = scale_table.reshape(NUM_FIELDS, NUM_SCALES * D)
    pad = jnp.zeros((NUM_FIELDS, STRIDE - R * D), jnp.float32)
    combined = jnp.concatenate([scale2, field_table, pad], axis=1).reshape(-1)
    ids = field_ids.astype(jnp.int32)
    mesh = plsc.VectorSubcoreMesh(core_axis_name="c", subcore_axis_name="s")
    k = pl.kernel(
        _body,
        out_type=jax.ShapeDtypeStruct((N * D,), jnp.float32),
        mesh=mesh,
        compiler_params=pltpu.CompilerParams(needs_layout_passes=False),
        scratch_types=[
            pltpu.VMEM((NUM_FIELDS * STRIDE,), jnp.float32),
            pltpu.VMEM((CHUNK,), jnp.int32),
            pltpu.VMEM((CHUNK,), jnp.float32),
            pltpu.VMEM((SUB * D,), jnp.float32),
        ],
    )
    out = k(combined, ids, values)
    return out.reshape(N, D)


# speedup vs baseline: 16.9669x; 2.1945x over previous
import functools

import jax
import jax.numpy as jnp
from jax import lax
from jax.experimental import pallas as pl
from jax.experimental.pallas import tpu as pltpu
from jax.experimental.pallas import tpu_sc as plsc

N = 106496
NUM_FIELDS = 100
D = 64
NUM_SCALES = 5
R = NUM_SCALES + 1
BLK = R * D
NW = 32
CHUNK = N // NW
SUB = 416
NSUB = CHUNK // SUB
GROUPS = SUB // 16
LANES = 16

_SF = [0.01, 0.1, 1.0, 10.0, 100.0]


def _body(table_hbm, ids_hbm, vals_hbm, out_hbm, table_v, ids_v, vals_v, out_v):
    wid = lax.axis_index("s") * 2 + lax.axis_index("c")
    base_tok = wid * CHUNK
    pltpu.sync_copy(table_hbm, table_v)
    pltpu.sync_copy(ids_hbm.at[pl.ds(base_tok, CHUNK)], ids_v)
    pltpu.sync_copy(vals_hbm.at[pl.ds(base_tok, CHUNK)], vals_v)

    def grp_body(s, g, _):
        t0 = s * SUB + g * LANES
        fvec = ids_v[pl.ds(t0, LANES)]
        vvec = vals_v[pl.ds(t0, LANES)]
        ws = []
        for r in range(NUM_SCALES):
            x = vvec * (2.0 * _SF[r])
            ws.append(1.0 - 2.0 / (jnp.exp(x) + 1.0))
        obase = g * (LANES * D)
        for t in range(LANES):
            base = fvec[t] * BLK
            rows = [table_v[pl.ds(base + k * LANES, LANES)] for k in range(BLK // LANES)]
            for j in range(D // LANES):
                acc = rows[NUM_SCALES * (D // LANES) + j]
                for r in range(NUM_SCALES):
                    acc = acc + ws[r][t] * rows[r * (D // LANES) + j]
                out_v[pl.ds(obase + t * D + j * LANES, LANES)] = acc
        return _

    def sub_body(s, _):
        lax.fori_loop(0, GROUPS, functools.partial(grp_body, s), 0)
        pltpu.sync_copy(out_v, out_hbm.at[pl.ds((base_tok + s * SUB) * D, SUB * D)])
        return _

    lax.fori_loop(0, NSUB, sub_body, 0)


def kernel(field_ids, values, field_table, scale_table):
    scale2 = scale_table.reshape(NUM_FIELDS, NUM_SCALES * D)
    combined = jnp.concatenate([scale2, field_table], axis=1).reshape(-1)
    ids = field_ids.astype(jnp.int32)
    mesh = plsc.VectorSubcoreMesh(core_axis_name="c", subcore_axis_name="s")
    k = pl.kernel(
        _body,
        out_type=jax.ShapeDtypeStruct((N * D,), jnp.float32),
        mesh=mesh,
        compiler_params=pltpu.CompilerParams(needs_layout_passes=False),
        scratch_types=[
            pltpu.VMEM((NUM_FIELDS * BLK,), jnp.float32),
            pltpu.VMEM((CHUNK,), jnp.int32),
            pltpu.VMEM((CHUNK,), jnp.float32),
            pltpu.VMEM((SUB * D,), jnp.float32),
        ],
    )
    out = k(combined, ids, values)
    return out.reshape(N, D)


# speedup vs baseline: 17.2524x; 1.0168x over previous
import functools

import jax
import jax.numpy as jnp
from jax import lax
from jax.experimental import pallas as pl
from jax.experimental.pallas import tpu as pltpu
from jax.experimental.pallas import tpu_sc as plsc

N = 106496
NUM_FIELDS = 100
D = 64
NUM_SCALES = 5
R = NUM_SCALES + 1
BLK = R * D
NW = 32
CHUNK = N // NW
SUB = 416
NSUB = CHUNK // SUB
GROUPS = SUB // 16
LANES = 16

_SF = [0.01, 0.1, 1.0, 10.0, 100.0]


def _body(table_hbm, ids_hbm, vals_hbm, out_hbm, table_v, ids_v, vals_v, out_v):
    wid = lax.axis_index("s") * 2 + lax.axis_index("c")
    base_tok = wid * CHUNK
    pltpu.sync_copy(table_hbm, table_v)
    pltpu.sync_copy(ids_hbm.at[pl.ds(base_tok, CHUNK)], ids_v)
    pltpu.sync_copy(vals_hbm.at[pl.ds(base_tok, CHUNK)], vals_v)

    def grp_body(s, g, _):
        t0 = s * SUB + g * LANES
        fvec = ids_v[pl.ds(t0, LANES)]
        vvec = vals_v[pl.ds(t0, LANES)]
        ws_u = []
        for r in range(NUM_SCALES):
            x = vvec * (2.0 * _SF[r])
            w = 1.0 - 2.0 / (jnp.exp(x) + 1.0)
            wp = plsc.pack(w, w, format=plsc.PackFormat.INTERLEAVED)
            ws_u.append(plsc.bitcast(wp, jnp.uint32))
        obase = g * (LANES * D)
        for t in range(LANES):
            base = fvec[t] * BLK
            rows = [table_v[pl.ds(base + k * 32, 32)] for k in range(BLK // 32)]
            wbfs = [
                plsc.bitcast(jnp.broadcast_to(ws_u[r][t], (LANES,)), jnp.bfloat16)
                for r in range(NUM_SCALES)
            ]
            for h in range(2):
                acc = rows[NUM_SCALES * 2 + h]
                for r in range(NUM_SCALES):
                    acc = acc + wbfs[r] * rows[r * 2 + h]
                a, b = plsc.unpack(acc, format=plsc.PackFormat.INTERLEAVED)
                out_v[pl.ds(obase + t * D + h * 32, LANES)] = a
                out_v[pl.ds(obase + t * D + h * 32 + LANES, LANES)] = b
        return _

    def sub_body(s, _):
        lax.fori_loop(0, GROUPS, functools.partial(grp_body, s), 0)
        pltpu.sync_copy(out_v, out_hbm.at[pl.ds((base_tok + s * SUB) * D, SUB * D)])
        return _

    lax.fori_loop(0, NSUB, sub_body, 0)


def kernel(field_ids, values, field_table, scale_table):
    scale2 = scale_table.reshape(NUM_FIELDS, NUM_SCALES * D)
    combined = jnp.concatenate([scale2, field_table], axis=1)
    t5 = combined.reshape(NUM_FIELDS, R, 2, 2, LANES)
    ti = t5.transpose(0, 1, 2, 4, 3)
    table_bf = ti.astype(jnp.bfloat16).reshape(-1)
    ids = field_ids.astype(jnp.int32)
    mesh = plsc.VectorSubcoreMesh(core_axis_name="c", subcore_axis_name="s")
    k = pl.kernel(
        _body,
        out_type=jax.ShapeDtypeStruct((N * D,), jnp.float32),
        mesh=mesh,
        compiler_params=pltpu.CompilerParams(needs_layout_passes=False),
        scratch_types=[
            pltpu.VMEM((NUM_FIELDS * BLK,), jnp.bfloat16),
            pltpu.VMEM((CHUNK,), jnp.int32),
            pltpu.VMEM((CHUNK,), jnp.float32),
            pltpu.VMEM((SUB * D,), jnp.float32),
        ],
    )
    out = k(table_bf, ids, values)
    return out.reshape(N, D)
